# Initial kernel scaffold; baseline (speedup 1.0000x reference)
import jax
import jax.numpy as jnp
from jax.experimental import pallas as pl


def kernel(inputs, table):
    raise NotImplementedError("write your pallas kernel here")


# speedup vs baseline: 3.9521x; 3.9521x over previous
import functools

import jax
import jax.numpy as jnp
from jax import lax
from jax.experimental import pallas as pl
from jax.experimental.pallas import tpu as pltpu
from jax.experimental.pallas import tpu_sc as plsc

_NUM_WORKERS = 32
_CHUNK = 512


@functools.partial(jax.jit, static_argnums=(2, 3))
def _sc_gather(flat_idx, table, n_total, chunk):
    emb = table.shape[1]
    b_per_w = n_total // _NUM_WORKERS
    n_chunks = b_per_w // chunk
    mesh = plsc.VectorSubcoreMesh(core_axis_name="c", subcore_axis_name="s")

    @functools.partial(
        pl.kernel,
        out_type=jax.ShapeDtypeStruct((n_total, emb), jnp.float32),
        mesh=mesh,
        scratch_types=[
            pltpu.VMEM((chunk,), jnp.int32),
            pltpu.VMEM((chunk, emb), jnp.float32),
            pltpu.SemaphoreType.DMA,
        ],
        compiler_params=pltpu.CompilerParams(use_tc_tiling_on_sc=False),
    )
    def k(idx_hbm, table_hbm, out_hbm, idx_v, rows_v, sem):
        wid = lax.axis_index("s") * 2 + lax.axis_index("c")
        base = wid * b_per_w

        def body(i, carry):
            off = base + i * chunk
            pltpu.sync_copy(idx_hbm.at[pl.ds(off, chunk)], idx_v)
            pltpu.async_copy(table_hbm.at[idx_v], rows_v, sem).wait()
            pltpu.sync_copy(rows_v, out_hbm.at[pl.ds(off, chunk)])
            return carry

        lax.fori_loop(0, n_chunks, body, 0)

    return k(flat_idx, table)


def kernel(inputs, table):
    batch, seq = inputs.shape
    emb = table.shape[1]
    flat_idx = inputs.reshape(-1).astype(jnp.int32)
    out = _sc_gather(flat_idx, table, batch * seq, _CHUNK)
    return out.reshape(batch, seq, emb)


# speedup vs baseline: 4.2167x; 1.0670x over previous
import functools

import jax
import jax.numpy as jnp
from jax import lax
from jax.experimental import pallas as pl
from jax.experimental.pallas import tpu as pltpu
from jax.experimental.pallas import tpu_sc as plsc

_NUM_WORKERS = 32
_CHUNK = 512


@functools.partial(jax.jit, static_argnums=(2, 3))
def _sc_gather(flat_idx, table, n_total, chunk):
    emb = table.shape[1]
    b_per_w = n_total // _NUM_WORKERS
    n_chunks = b_per_w // chunk
    assert n_chunks % 2 == 0
    mesh = plsc.VectorSubcoreMesh(core_axis_name="c", subcore_axis_name="s")

    @functools.partial(
        pl.kernel,
        out_type=jax.ShapeDtypeStruct((n_total, emb), jnp.float32),
        mesh=mesh,
        scratch_types=[
            pltpu.VMEM((b_per_w,), jnp.int32),
            pltpu.VMEM((chunk, emb), jnp.float32),
            pltpu.VMEM((chunk, emb), jnp.float32),
            pltpu.SemaphoreType.DMA,
            pltpu.SemaphoreType.DMA,
            pltpu.SemaphoreType.DMA,
            pltpu.SemaphoreType.DMA,
        ],
        compiler_params=pltpu.CompilerParams(use_tc_tiling_on_sc=False),
    )
    def k(idx_hbm, table_hbm, out_hbm, idx_v, buf_a, buf_b, gs_a, gs_b, ss_a, ss_b):
        wid = lax.axis_index("s") * 2 + lax.axis_index("c")
        base = wid * b_per_w
        pltpu.sync_copy(idx_hbm.at[pl.ds(base, b_per_w)], idx_v)

        def gather(c, buf, sem):
            return pltpu.make_async_copy(
                table_hbm.at[idx_v.at[pl.ds(c * chunk, chunk)]], buf, sem)

        def store(c, buf, sem):
            return pltpu.make_async_copy(
                buf, out_hbm.at[pl.ds(base + c * chunk, chunk)], sem)

        def body(j, carry):
            c0, c1 = 2 * j, 2 * j + 1

            @pl.when(j > 0)
            def _():
                store(c0, buf_a, ss_a).wait()

            gather(c0, buf_a, gs_a).start()

            @pl.when(j > 0)
            def _():
                store(c1, buf_b, ss_b).wait()

            gather(c1, buf_b, gs_b).start()

            gather(c0, buf_a, gs_a).wait()
            store(c0, buf_a, ss_a).start()
            gather(c1, buf_b, gs_b).wait()
            store(c1, buf_b, ss_b).start()
            return carry

        lax.fori_loop(0, n_chunks // 2, body, 0)
        store(0, buf_a, ss_a).wait()
        store(0, buf_b, ss_b).wait()

    return k(flat_idx, table)


def kernel(inputs, table):
    batch, seq = inputs.shape
    emb = table.shape[1]
    flat_idx = inputs.reshape(-1).astype(jnp.int32)
    out = _sc_gather(flat_idx, table, batch * seq, _CHUNK)
    return out.reshape(batch, seq, emb)


# speedup vs baseline: 4.2319x; 1.0036x over previous
import functools

import jax
import jax.numpy as jnp
from jax import lax
from jax.experimental import pallas as pl
from jax.experimental.pallas import tpu as pltpu
from jax.experimental.pallas import tpu_sc as plsc

_NUM_WORKERS = 32
_NBUF = 4


@jax.jit
def _sc_gather(idx, table):
    batch, seq = idx.shape
    emb = table.shape[1]
    rows_per_w = batch // _NUM_WORKERS
    assert rows_per_w % _NBUF == 0
    mesh = plsc.VectorSubcoreMesh(core_axis_name="c", subcore_axis_name="s")

    @functools.partial(
        pl.kernel,
        out_type=jax.ShapeDtypeStruct((batch, seq, emb), jnp.float32),
        mesh=mesh,
        scratch_types=[
            pltpu.VMEM((rows_per_w, seq), jnp.int32),
            *[pltpu.VMEM((seq, emb), jnp.float32) for _ in range(_NBUF)],
            *[pltpu.SemaphoreType.DMA for _ in range(2 * _NBUF)],
        ],
        compiler_params=pltpu.CompilerParams(use_tc_tiling_on_sc=False),
    )
    def k(idx_hbm, table_hbm, out_hbm, idx_v, *bufs_sems):
        bufs = bufs_sems[:_NBUF]
        gsems = bufs_sems[_NBUF:2 * _NBUF]
        ssems = bufs_sems[2 * _NBUF:]
        wid = lax.axis_index("s") * 2 + lax.axis_index("c")
        row0 = wid * rows_per_w
        pltpu.sync_copy(idx_hbm.at[pl.ds(row0, rows_per_w)], idx_v)

        def gather(r, j):
            return pltpu.make_async_copy(
                table_hbm.at[idx_v.at[r]], bufs[j], gsems[j])

        def store(r, j):
            return pltpu.make_async_copy(bufs[j], out_hbm.at[row0 + r], ssems[j])

        def body(o, carry):
            for j in range(_NBUF):
                r = o * _NBUF + j

                @pl.when(o > 0)
                def _(j=j, r=r):
                    store(r, j).wait()

                gather(r, j).start()
            for j in range(_NBUF):
                r = o * _NBUF + j
                gather(r, j).wait()
                store(r, j).start()
            return carry

        lax.fori_loop(0, rows_per_w // _NBUF, body, 0)
        for j in range(_NBUF):
            store(0, j).wait()

    return k(idx, table)


def kernel(inputs, table):
    return _sc_gather(inputs.astype(jnp.int32), table)


# speedup vs baseline: 4.4058x; 1.0411x over previous
import functools

import jax
import jax.numpy as jnp
from jax import lax
from jax.experimental import pallas as pl
from jax.experimental.pallas import tpu as pltpu
from jax.experimental.pallas import tpu_sc as plsc

_NUM_WORKERS = 32
_NBUF = 2
_LANES = 128


@jax.jit
def _sc_gather(idx, table128):
    batch, seq = idx.shape
    emb = 64
    rows_per_w = batch // _NUM_WORKERS
    assert rows_per_w % _NBUF == 0
    mesh = plsc.VectorSubcoreMesh(core_axis_name="c", subcore_axis_name="s")

    @functools.partial(
        pl.kernel,
        out_type=jax.ShapeDtypeStruct((batch, seq, emb), jnp.float32),
        mesh=mesh,
        scratch_types=[
            *[pltpu.VMEM((seq,), jnp.int32) for _ in range(_NBUF)],
            *[pltpu.VMEM((seq, _LANES), jnp.float32) for _ in range(_NBUF)],
            *[pltpu.VMEM((seq, emb), jnp.float32) for _ in range(_NBUF)],
            *[pltpu.SemaphoreType.DMA for _ in range(3 * _NBUF)],
        ],
    )
    def k(idx_hbm, table_hbm, out_hbm, *scratch):
        idxb = scratch[:_NBUF]
        wide = scratch[_NBUF:2 * _NBUF]
        narrow = scratch[2 * _NBUF:3 * _NBUF]
        isems = scratch[3 * _NBUF:4 * _NBUF]
        gsems = scratch[4 * _NBUF:5 * _NBUF]
        ssems = scratch[5 * _NBUF:]
        wid = lax.axis_index("s") * 2 + lax.axis_index("c")
        row0 = wid * rows_per_w

        def idxcopy(r, j):
            return pltpu.make_async_copy(idx_hbm.at[row0 + r], idxb[j], isems[j])

        def gather(j):
            return pltpu.make_async_copy(
                table_hbm.at[idxb[j]], wide[j], gsems[j])

        def store(r, j):
            return pltpu.make_async_copy(
                narrow[j], out_hbm.at[row0 + r], ssems[j])

        def compact(j):
            w, n = wide[j], narrow[j]

            def crow(i, carry):
                for rr in range(4):
                    for kk in range(4):
                        sl = pl.ds(16 * kk, 16)
                        n[4 * i + rr, sl] = w[4 * i + rr, sl]
                return carry

            lax.fori_loop(0, seq // 4, crow, 0)

        def body(o, carry):
            for j in range(_NBUF):
                r = o * _NBUF + j

                @pl.when(o > 0)
                def _(j=j, r=r):
                    store(r, j).wait()

                idxcopy(r, j).start()
            for j in range(_NBUF):
                idxcopy(0, j).wait()
                gather(j).start()
            for j in range(_NBUF):
                r = o * _NBUF + j
                gather(j).wait()
                compact(j)
                store(r, j).start()
            return carry

        lax.fori_loop(0, rows_per_w // _NBUF, body, 0)
        for j in range(_NBUF):
            store(0, j).wait()

    return k(idx, table128)


def kernel(inputs, table):
    vocab, emb = table.shape
    table128 = jnp.pad(table, ((0, 0), (0, _LANES - emb)))
    return _sc_gather(inputs.astype(jnp.int32), table128)


# speedup vs baseline: 6.0918x; 1.3827x over previous
import functools

import jax
import jax.numpy as jnp
from jax import lax
from jax.experimental import pallas as pl
from jax.experimental.pallas import tpu as pltpu
from jax.experimental.pallas import tpu_sc as plsc

_NUM_WORKERS = 32
_UNROLL = 8


@jax.jit
def _sc_gather_t(idx_t, table_t):
    seq, batch = idx_t.shape
    emb, vocab = table_t.shape
    cpasses = emb // _NUM_WORKERS
    mesh = plsc.VectorSubcoreMesh(core_axis_name="c", subcore_axis_name="s")

    @functools.partial(
        pl.kernel,
        out_type=jax.ShapeDtypeStruct((seq, emb, batch), jnp.float32),
        mesh=mesh,
        scratch_types=[
            pltpu.VMEM((vocab,), jnp.float32),
            pltpu.VMEM((batch,), jnp.int32),
            pltpu.VMEM((batch,), jnp.int32),
            pltpu.VMEM((batch,), jnp.float32),
            pltpu.VMEM((batch,), jnp.float32),
            *[pltpu.SemaphoreType.DMA for _ in range(4)],
        ],
        compiler_params=pltpu.CompilerParams(needs_layout_passes=False),
    )
    def k(idx_hbm, tab_hbm, out_hbm, tab_v, ib0, ib1, ob0, ob1,
          is0, is1, os0, os1):
        ibs, obs = (ib0, ib1), (ob0, ob1)
        isems, osems = (is0, is1), (os0, os1)
        wid = lax.axis_index("s") * 2 + lax.axis_index("c")

        def idxcopy(s, j):
            return pltpu.make_async_copy(idx_hbm.at[s], ibs[j], isems[j])

        def ostore(s, c, j):
            return pltpu.make_async_copy(obs[j], out_hbm.at[s, c], osems[j])

        def compute(j):
            ib, ob = ibs[j], obs[j]

            def inner(kk, carry):
                for u in range(_UNROLL):
                    sl = pl.ds((kk * _UNROLL + u) * 16, 16)
                    ob[sl] = plsc.load_gather(tab_v, [ib[sl]])
                return carry

            lax.fori_loop(0, batch // (16 * _UNROLL), inner, 0)

        for cpass in range(cpasses):
            c = wid + _NUM_WORKERS * cpass
            pltpu.sync_copy(tab_hbm.at[c], tab_v)
            idxcopy(0, 0).start()

            def body(so, carry, c=c):
                for j in range(2):
                    s = 2 * so + j

                    @pl.when(s + 1 < seq)
                    def _(s=s, j=j):
                        idxcopy(s + 1, 1 - j).start()

                    idxcopy(s, j).wait()

                    @pl.when(s >= 2)
                    def _(c=c, j=j):
                        ostore(0, c, j).wait()

                    compute(j)
                    ostore(s, c, j).start()
                return carry

            lax.fori_loop(0, seq // 2, body, 0)
            for j in range(2):
                ostore(0, c, j).wait()

    return k(idx_t, table_t)


def kernel(inputs, table):
    out_t = _sc_gather_t(inputs.T.astype(jnp.int32), table.T)
    return out_t.transpose(2, 0, 1)


# speedup vs baseline: 9.0128x; 1.4795x over previous
import functools

import jax
import jax.numpy as jnp
from jax import lax
from jax.experimental import pallas as pl
from jax.experimental.pallas import tpu as pltpu
from jax.experimental.pallas import tpu_sc as plsc

_NUM_WORKERS = 32
_UNROLL = 8


@jax.jit
def _sc_gather_t(idx_t, table_t):
    seq, batch = idx_t.shape
    emb, vocab = table_t.shape
    cpasses = emb // _NUM_WORKERS
    mesh = plsc.VectorSubcoreMesh(core_axis_name="c", subcore_axis_name="s")

    @functools.partial(
        pl.kernel,
        out_type=jax.ShapeDtypeStruct((seq, emb, batch), jnp.float32),
        mesh=mesh,
        scratch_types=[
            pltpu.VMEM((vocab,), jnp.float32),
            pltpu.VMEM((batch,), jnp.int32),
            pltpu.VMEM((batch,), jnp.int32),
            pltpu.VMEM((batch,), jnp.float32),
            pltpu.VMEM((batch,), jnp.float32),
            *[pltpu.SemaphoreType.DMA for _ in range(4)],
        ],
        compiler_params=pltpu.CompilerParams(needs_layout_passes=False),
    )
    def k(idx_hbm, tab_hbm, out_hbm, tab_v, ib0, ib1, ob0, ob1,
          is0, is1, os0, os1):
        ibs, obs = (ib0, ib1), (ob0, ob1)
        isems, osems = (is0, is1), (os0, os1)
        wid = lax.axis_index("s") * 2 + lax.axis_index("c")

        def idxcopy(s, j):
            return pltpu.make_async_copy(idx_hbm.at[s], ibs[j], isems[j])

        def ostore(s, c, j):
            return pltpu.make_async_copy(obs[j], out_hbm.at[s, c], osems[j])

        def compute(j):
            ib, ob = ibs[j], obs[j]

            @plsc.parallel_loop(0, batch // 16, unroll=_UNROLL)
            def _(kk):
                sl = pl.ds(kk * 16, 16)
                ob[sl] = plsc.load_gather(tab_v, [ib[sl]])

        for cpass in range(cpasses):
            c = wid + _NUM_WORKERS * cpass
            pltpu.sync_copy(tab_hbm.at[c], tab_v)
            idxcopy(0, 0).start()

            def body(so, carry, c=c):
                for j in range(2):
                    s = 2 * so + j

                    @pl.when(s + 1 < seq)
                    def _(s=s, j=j):
                        idxcopy(s + 1, 1 - j).start()

                    idxcopy(s, j).wait()

                    @pl.when(s >= 2)
                    def _(c=c, j=j):
                        ostore(0, c, j).wait()

                    compute(j)
                    ostore(s, c, j).start()
                return carry

            lax.fori_loop(0, seq // 2, body, 0)
            for j in range(2):
                ostore(0, c, j).wait()

    return k(idx_t, table_t)


def kernel(inputs, table):
    out_t = _sc_gather_t(inputs.T.astype(jnp.int32), table.T)
    return out_t.transpose(2, 0, 1)


# speedup vs baseline: 9.0538x; 1.0045x over previous
import functools

import jax
import jax.numpy as jnp
from jax import lax
from jax.experimental import pallas as pl
from jax.experimental.pallas import tpu as pltpu
from jax.experimental.pallas import tpu_sc as plsc

_NUM_WORKERS = 32
_UNROLL = 16


@jax.jit
def _sc_gather_t(idx_t, table_t):
    seq, batch = idx_t.shape
    emb, vocab = table_t.shape
    cpasses = emb // _NUM_WORKERS
    mesh = plsc.VectorSubcoreMesh(core_axis_name="c", subcore_axis_name="s")

    @functools.partial(
        pl.kernel,
        out_type=jax.ShapeDtypeStruct((seq, emb, batch), jnp.float32),
        mesh=mesh,
        scratch_types=[
            pltpu.VMEM((vocab,), jnp.float32),
            pltpu.VMEM((batch,), jnp.int32),
            pltpu.VMEM((batch,), jnp.int32),
            pltpu.VMEM((batch,), jnp.float32),
            pltpu.VMEM((batch,), jnp.float32),
            *[pltpu.SemaphoreType.DMA for _ in range(4)],
        ],
        compiler_params=pltpu.CompilerParams(needs_layout_passes=False),
    )
    def k(idx_hbm, tab_hbm, out_hbm, tab_v, ib0, ib1, ob0, ob1,
          is0, is1, os0, os1):
        ibs, obs = (ib0, ib1), (ob0, ob1)
        isems, osems = (is0, is1), (os0, os1)
        wid = lax.axis_index("s") * 2 + lax.axis_index("c")

        def idxcopy(s, j):
            return pltpu.make_async_copy(idx_hbm.at[s], ibs[j], isems[j])

        def ostore(s, c, j):
            return pltpu.make_async_copy(obs[j], out_hbm.at[s, c], osems[j])

        def compute(j):
            ib, ob = ibs[j], obs[j]

            @plsc.parallel_loop(0, batch // 16, unroll=_UNROLL)
            def _(kk):
                sl = pl.ds(kk * 16, 16)
                ob[sl] = plsc.load_gather(tab_v, [ib[sl]])

        for cpass in range(cpasses):
            c = wid + _NUM_WORKERS * cpass
            pltpu.sync_copy(tab_hbm.at[c], tab_v)
            idxcopy(0, 0).start()

            def body(so, carry, c=c):
                for j in range(2):
                    s = 2 * so + j

                    @pl.when(s + 1 < seq)
                    def _(s=s, j=j):
                        idxcopy(s + 1, 1 - j).start()

                    idxcopy(s, j).wait()

                    @pl.when(s >= 2)
                    def _(c=c, j=j):
                        ostore(0, c, j).wait()

                    compute(j)
                    ostore(s, c, j).start()
                return carry

            lax.fori_loop(0, seq // 2, body, 0)
            for j in range(2):
                ostore(0, c, j).wait()

    return k(idx_t, table_t)


def kernel(inputs, table):
    out_t = _sc_gather_t(inputs.T.astype(jnp.int32), table.T)
    return out_t.transpose(2, 0, 1)


# speedup vs baseline: 9.0580x; 1.0005x over previous
import functools

import jax
import jax.numpy as jnp
from jax import lax
from jax.experimental import pallas as pl
from jax.experimental.pallas import tpu as pltpu
from jax.experimental.pallas import tpu_sc as plsc

_NUM_WORKERS = 32
_UNROLL = 16


@functools.partial(jax.jit, static_argnums=(2, 3))
def _sc_gather_t(idx_flat, table_t, seq, batch):
    emb, vocab = table_t.shape
    cpasses = emb // _NUM_WORKERS
    mesh = plsc.VectorSubcoreMesh(core_axis_name="c", subcore_axis_name="s")

    @functools.partial(
        pl.kernel,
        out_type=jax.ShapeDtypeStruct((seq, emb, batch), jnp.float32),
        mesh=mesh,
        scratch_types=[
            pltpu.VMEM((vocab,), jnp.float32),
            pltpu.VMEM((batch,), jnp.int32),
            pltpu.VMEM((batch,), jnp.int32),
            pltpu.VMEM((batch,), jnp.float32),
            pltpu.VMEM((batch,), jnp.float32),
            *[pltpu.SemaphoreType.DMA for _ in range(4)],
        ],
        compiler_params=pltpu.CompilerParams(needs_layout_passes=False),
    )
    def k(idx_hbm, tab_hbm, out_hbm, tab_v, ib0, ib1, ob0, ob1,
          is0, is1, os0, os1):
        ibs, obs = (ib0, ib1), (ob0, ob1)
        isems, osems = (is0, is1), (os0, os1)
        wid = lax.axis_index("s") * 2 + lax.axis_index("c")

        def idxcopy(s, j):
            return pltpu.make_async_copy(
                idx_hbm.at[pl.ds(s * batch, batch)], ibs[j], isems[j])

        def ostore(s, c, j):
            return pltpu.make_async_copy(obs[j], out_hbm.at[s, c], osems[j])

        def compute(j):
            ib, ob = ibs[j], obs[j]

            @plsc.parallel_loop(0, batch // 16, unroll=_UNROLL)
            def _(kk):
                sl = pl.ds(kk * 16, 16)
                ob[sl] = plsc.load_gather(tab_v, [ib[sl]])

        for cpass in range(cpasses):
            c = wid + _NUM_WORKERS * cpass
            pltpu.sync_copy(tab_hbm.at[c], tab_v)
            idxcopy(0, 0).start()

            def body(so, carry, c=c):
                for j in range(2):
                    s = 2 * so + j

                    @pl.when(s + 1 < seq)
                    def _(s=s, j=j):
                        idxcopy(s + 1, 1 - j).start()

                    idxcopy(s, j).wait()

                    @pl.when(s >= 2)
                    def _(c=c, j=j):
                        ostore(0, c, j).wait()

                    compute(j)
                    ostore(s, c, j).start()
                return carry

            lax.fori_loop(0, seq // 2, body, 0)
            for j in range(2):
                ostore(0, c, j).wait()

    return k(idx_flat, table_t)


def kernel(inputs, table):
    batch, seq = inputs.shape
    idx_flat = inputs.T.reshape(-1).astype(jnp.int32)
    out_t = _sc_gather_t(idx_flat, table.T, seq, batch)
    return out_t.transpose(2, 0, 1)
